# baseline (speedup 1.0000x reference)
import jax
import jax.numpy as jnp
from jax import lax
from jax.experimental import pallas as pl
from jax.experimental.pallas import tpu as pltpu
from jax.experimental.pallas import tpu_sc as plsc

NC = 2
NS = 16
L = 16
NW = NC * NS

B = 64
N = 80000
MAXLEN = 120000
G_PER_ROW = MAXLEN // L
ROWS_PER_W = B // NW
CHO = 20000
NCHUNK = N // CHO
CG = CHO // L
CGP = 1256


def _sc_body(wav_hbm, noise_hbm, gidx_hbm, meta_hbm, out_hbm,
             idx_v, meta_v, nbuf, wbuf, obuf, sem):
    wid = lax.axis_index("s") * NC + lax.axis_index("c")

    pltpu.sync_copy(meta_hbm, meta_v)
    rvec = meta_v[...]
    base_vec = rvec + lax.iota(jnp.int32, 16)

    for t in range(ROWS_PER_W):
        i = wid * ROWS_PER_W + t
        pltpu.sync_copy(gidx_hbm.at[pl.ds(i * NCHUNK * CGP, NCHUNK * CGP)],
                        idx_v)
        for c in range(NCHUNK):
            pltpu.async_copy(
                noise_hbm.at[idx_v.at[pl.ds(c * CGP, CGP)]], nbuf, sem
            ).wait()
            pltpu.sync_copy(wav_hbm.at[pl.ds(i * N + c * CHO, CHO)], wbuf)

            def body(j, _):
                v = base_vec + j * L
                nv = plsc.load_gather(nbuf, [v >> 4, v & 15])
                wv = wbuf[pl.ds(j * L, L)]
                res = jnp.float32(0.05) * nv + wv
                res = jnp.minimum(jnp.maximum(res, jnp.float32(-1.0)),
                                  jnp.float32(1.0))
                obuf[pl.ds(j * L, L)] = res
                return _

            lax.fori_loop(0, CHO // L, body, None)
            pltpu.sync_copy(obuf, out_hbm.at[pl.ds(i * N + c * CHO, CHO)])


@jax.jit
def _add_noise_sc(waveforms, noise_files, gidx, meta):
    mesh = plsc.VectorSubcoreMesh(
        core_axis_name="c", subcore_axis_name="s",
        num_cores=NC, num_subcores=NS,
    )
    fn = pl.kernel(
        _sc_body,
        out_type=jax.ShapeDtypeStruct((B * N,), jnp.float32),
        mesh=mesh,
        scratch_types=[
            pltpu.VMEM((NCHUNK * CGP,), jnp.int32),
            pltpu.VMEM((L,), jnp.int32),
            pltpu.VMEM((CGP, L), jnp.float32),
            pltpu.VMEM((CHO,), jnp.float32),
            pltpu.VMEM((CHO,), jnp.float32),
            pltpu.SemaphoreType.DMA,
        ],
        compiler_params=pltpu.CompilerParams(
            needs_layout_passes=False, use_tc_tiling_on_sc=False),
    )
    out = fn(waveforms.reshape(B * N),
             noise_files.reshape(200 * G_PER_ROW, L), gidx, meta)
    return out.reshape(B, N)


def kernel(waveforms, lengths, noise_files, random_index, start_index):
    del lengths
    ridx = random_index.astype(jnp.int32)
    start = start_index.astype(jnp.int32)
    r = start & 15
    s_g = start >> 4
    karange = jnp.minimum(jnp.arange(NCHUNK)[:, None] * CG
                          + jnp.arange(CGP)[None, :], N // L)
    gidx = (ridx[:, None, None] * G_PER_ROW + s_g) + karange[None]
    meta = jnp.full((L,), r, dtype=jnp.int32)
    return _add_noise_sc(waveforms, noise_files,
                         gidx.reshape(-1).astype(jnp.int32), meta)


# speedup vs baseline: 1.4160x; 1.4160x over previous
import jax
import jax.numpy as jnp
from jax import lax
from jax.experimental import pallas as pl
from jax.experimental.pallas import tpu as pltpu
from jax.experimental.pallas import tpu_sc as plsc

NC = 2
NS = 16
L = 16
NW = NC * NS

B = 64
N = 80000
MAXLEN = 120000
G_PER_ROW = MAXLEN // L
ROWS_PER_W = B // NW
CHO = 16000
NCHUNK = N // CHO
CG = CHO // L
CGP = 1008
NSTEP = ROWS_PER_W * NCHUNK


def _sc_body(wav_hbm, noise_hbm, gidx_hbm, meta_hbm, out_hbm,
             idx_v, meta_v, nbuf, wbuf, obuf,
             gsem, wsem, osem):
    wid = lax.axis_index("s") * NC + lax.axis_index("c")
    i0 = wid * ROWS_PER_W

    pltpu.sync_copy(meta_hbm, meta_v)
    pltpu.sync_copy(
        gidx_hbm.at[pl.ds(i0 * NCHUNK * CGP, NSTEP * CGP)], idx_v)

    rvec = meta_v[...]
    base_vec = rvec + lax.iota(jnp.int32, 16)
    gvec = base_vec >> 4
    lvec = base_vec & 15

    def start_in(k):
        p = k % 2
        pltpu.async_copy(
            noise_hbm.at[idx_v.at[pl.ds(k * CGP, CGP)]], nbuf.at[p], gsem[p])
        t, c = divmod(k, NCHUNK)
        off = (i0 + t) * N + c * CHO
        pltpu.async_copy(wav_hbm.at[pl.ds(off, CHO)], wbuf.at[p], wsem[p])

    start_in(0)
    for k in range(NSTEP):
        p = k % 2
        if k + 1 < NSTEP:
            start_in(k + 1)
        pltpu.make_async_copy(noise_hbm.at[idx_v.at[pl.ds(k * CGP, CGP)]],
                              nbuf.at[p], gsem[p]).wait()
        t, c = divmod(k, NCHUNK)
        off = (i0 + t) * N + c * CHO
        pltpu.make_async_copy(wav_hbm.at[pl.ds(off, CHO)], wbuf.at[p],
                              wsem[p]).wait()
        if k >= 2:
            pltpu.make_async_copy(obuf.at[p],
                                  out_hbm.at[pl.ds(off, CHO)], osem[p]).wait()

        @plsc.parallel_loop(0, CG, 1, unroll=8)
        def body(j):
            nv = plsc.load_gather(nbuf.at[p], [gvec + j, lvec])
            wv = wbuf[p, pl.ds(j * L, L)]
            res = jnp.float32(0.05) * nv + wv
            res = jnp.minimum(jnp.maximum(res, jnp.float32(-1.0)),
                              jnp.float32(1.0))
            obuf[p, pl.ds(j * L, L)] = res

        pltpu.async_copy(obuf.at[p], out_hbm.at[pl.ds(off, CHO)], osem[p])

    for k in (NSTEP - 2, NSTEP - 1):
        p = k % 2
        t, c = divmod(k, NCHUNK)
        off = (i0 + t) * N + c * CHO
        pltpu.make_async_copy(obuf.at[p], out_hbm.at[pl.ds(off, CHO)],
                              osem[p]).wait()


@jax.jit
def _add_noise_sc(waveforms, noise_files, gidx, meta):
    mesh = plsc.VectorSubcoreMesh(
        core_axis_name="c", subcore_axis_name="s",
        num_cores=NC, num_subcores=NS,
    )
    fn = pl.kernel(
        _sc_body,
        out_type=jax.ShapeDtypeStruct((B * N,), jnp.float32),
        mesh=mesh,
        scratch_types=[
            pltpu.VMEM((NSTEP * CGP,), jnp.int32),
            pltpu.VMEM((L,), jnp.int32),
            pltpu.VMEM((2, CGP, L), jnp.float32),
            pltpu.VMEM((2, CHO), jnp.float32),
            pltpu.VMEM((2, CHO), jnp.float32),
            [pltpu.SemaphoreType.DMA] * 2,
            [pltpu.SemaphoreType.DMA] * 2,
            [pltpu.SemaphoreType.DMA] * 2,
        ],
        compiler_params=pltpu.CompilerParams(
            needs_layout_passes=False, use_tc_tiling_on_sc=False),
    )
    out = fn(waveforms.reshape(B * N),
             noise_files.reshape(200 * G_PER_ROW, L), gidx, meta)
    return out.reshape(B, N)


def kernel(waveforms, lengths, noise_files, random_index, start_index):
    del lengths
    ridx = random_index.astype(jnp.int32)
    start = start_index.astype(jnp.int32)
    r = start & 15
    s_g = start >> 4
    karange = jnp.minimum(jnp.arange(NCHUNK)[:, None] * CG
                          + jnp.arange(CGP)[None, :], N // L)
    gidx = (ridx[:, None, None] * G_PER_ROW + s_g) + karange[None]
    meta = jnp.full((L,), r, dtype=jnp.int32)
    return _add_noise_sc(waveforms, noise_files,
                         gidx.reshape(-1).astype(jnp.int32), meta)


# speedup vs baseline: 5.8742x; 4.1484x over previous
import jax
import jax.numpy as jnp
from jax import lax
from jax.experimental import pallas as pl
from jax.experimental.pallas import tpu as pltpu
from jax.experimental.pallas import tpu_sc as plsc

NC = 2
NS = 16
L = 16
NW = NC * NS

B = 64
N = 80000
MAXLEN = 120000
ROWS_PER_W = B // NW
CHO = 16000
NCHUNK = N // CHO
PAD = 128
NSTEP = ROWS_PER_W * NCHUNK


def _sc_body(wav_hbm, noise_hbm, meta_hbm, out_hbm,
             nidx_v, widx_v, meta_v, nbuf, wbuf, obuf,
             gsem, wsem, osem):
    wid = lax.axis_index("s") * NC + lax.axis_index("c")

    pltpu.sync_copy(meta_hbm.at[pl.ds(L * wid, L)], nidx_v)
    pltpu.sync_copy(meta_hbm.at[pl.ds(NW * L + L + L * wid, L)], widx_v)
    pltpu.sync_copy(meta_hbm.at[pl.ds(NW * L, L)], meta_v)

    start_s = jnp.max(meta_v[...])
    c0 = pl.multiple_of(start_s & jnp.int32(-128), 128)
    r = start_s & jnp.int32(127)

    def start_in(k):
        p = k % 2
        t, c = divmod(k, NCHUNK)
        pltpu.async_copy(
            noise_hbm.at[nidx_v.at[pl.ds(8 * t, 1)],
                         pl.ds(c0 + c * CHO, CHO + PAD)],
            nbuf.at[p], gsem[p])
        pltpu.async_copy(
            wav_hbm.at[widx_v.at[pl.ds(8 * t, 1)], pl.ds(c * CHO, CHO)],
            wbuf.at[p], wsem[p])

    def wait_in(k):
        p = k % 2
        t, c = divmod(k, NCHUNK)
        pltpu.make_async_copy(
            noise_hbm.at[nidx_v.at[pl.ds(8 * t, 1)],
                         pl.ds(c0 + c * CHO, CHO + PAD)],
            nbuf.at[p], gsem[p]).wait()
        pltpu.make_async_copy(
            wav_hbm.at[widx_v.at[pl.ds(8 * t, 1)], pl.ds(c * CHO, CHO)],
            wbuf.at[p], wsem[p]).wait()

    def out_copy(k):
        p = k % 2
        t, c = divmod(k, NCHUNK)
        return pltpu.make_async_copy(
            obuf.at[p],
            out_hbm.at[widx_v.at[pl.ds(8 * t, 1)], pl.ds(c * CHO, CHO)],
            osem[p])

    start_in(0)
    for k in range(NSTEP):
        p = k % 2
        if k + 1 < NSTEP:
            start_in(k + 1)
        wait_in(k)
        if k >= 2:
            out_copy(k - 2).wait()

        @plsc.parallel_loop(0, CHO // L, 1, unroll=8)
        def body(j):
            nv = nbuf[p, 0, pl.ds(r + j * L, L)]
            wv = wbuf[p, 0, pl.ds(j * L, L)]
            res = jnp.float32(0.05) * nv + wv
            res = jnp.minimum(jnp.maximum(res, jnp.float32(-1.0)),
                              jnp.float32(1.0))
            obuf[p, 0, pl.ds(j * L, L)] = res

        out_copy(k).start()

    out_copy(NSTEP - 2).wait()
    out_copy(NSTEP - 1).wait()


@jax.jit
def _add_noise_sc(waveforms, noise_files, meta):
    mesh = plsc.VectorSubcoreMesh(
        core_axis_name="c", subcore_axis_name="s",
        num_cores=NC, num_subcores=NS,
    )
    fn = pl.kernel(
        _sc_body,
        out_type=jax.ShapeDtypeStruct((B, N), jnp.float32),
        mesh=mesh,
        scratch_types=[
            pltpu.VMEM((L,), jnp.int32),
            pltpu.VMEM((L,), jnp.int32),
            pltpu.VMEM((L,), jnp.int32),
            pltpu.VMEM((2, 1, CHO + PAD), jnp.float32),
            pltpu.VMEM((2, 1, CHO), jnp.float32),
            pltpu.VMEM((2, 1, CHO), jnp.float32),
            [pltpu.SemaphoreType.DMA] * 2,
            [pltpu.SemaphoreType.DMA] * 2,
            [pltpu.SemaphoreType.DMA] * 2,
        ],
        compiler_params=pltpu.CompilerParams(needs_layout_passes=False),
    )
    return fn(waveforms, noise_files, meta)


def kernel(waveforms, lengths, noise_files, random_index, start_index):
    del lengths
    ridx = random_index.astype(jnp.int32)
    start = start_index.astype(jnp.int32)
    npairs = jnp.zeros((NW, L), jnp.int32)
    npairs = npairs.at[:, 0].set(ridx[0::2]).at[:, 8].set(ridx[1::2])
    wi = jnp.arange(NW, dtype=jnp.int32) * 2
    wpairs = jnp.zeros((NW, L), jnp.int32)
    wpairs = wpairs.at[:, 0].set(wi).at[:, 8].set(wi + 1)
    meta = jnp.concatenate(
        [npairs.reshape(-1), jnp.full((L,), start, jnp.int32),
         wpairs.reshape(-1)])
    return _add_noise_sc(waveforms, noise_files, meta)
